# baseline (speedup 1.0000x reference)
import jax
import jax.numpy as jnp
from jax import lax
from jax.experimental import pallas as pl
from jax.experimental.pallas import tpu as pltpu
from jax.experimental.pallas import tpu_sc as plsc

N_NODES = 50000
N_EDGES = 800000
NUM_TYPES = 32
NUM_BASIS = 8
R_MAX = 5.0
OUT_W = NUM_BASIS + 2 * NUM_TYPES
OB_W = OUT_W + 1

NC, NS, L = 2, 16, 16
NW = NC * NS
B = 640
NCHUNK = N_EDGES // B
GROUPS = B // L
GEOM_W = 16

_TWO_PI = 6.283185307179586
_INV_TWO_PI = 1.0 / _TWO_PI
_SIN_C = (9.9998458677e-01, -1.6663258204e-01, 8.3123829338e-03,
          -1.9316182196e-04, 2.1732100681e-06)
_COS_C = (9.9999944342e-01, -4.9999558037e-01, 4.1661031574e-02,
          -1.3862743260e-03, 2.4253137751e-05, -2.2193694177e-07)


def _rsqrt(l2):
    i = plsc.bitcast(l2, jnp.int32)
    y = plsc.bitcast(jnp.int32(0x5F3759DF) - (i >> 1), jnp.float32)
    for _ in range(3):
        y = y * (1.5 - 0.5 * l2 * y * y)
    return y


def _sc_kernel(geom_hbm, src_hbm, dst_hbm, te0_hbm, te1_hbm, w_hbm, out_hbm,
               gsrc, gdst, wv, obuf, otc, otn, ats_v, atd_v, sem, sem2, isrc, idst):
    wid = lax.axis_index("s") * NC + lax.axis_index("c")
    pltpu.sync_copy(w_hbm, wv)
    iota = lax.iota(jnp.int32, L)

    def full_i(v):
        return jnp.full((L,), v, jnp.int32)

    @pl.loop(wid, NCHUNK, step=NW)
    def _chunk(c):
        base = c * B
        pltpu.sync_copy(src_hbm.at[pl.ds(base, B)], isrc)
        pltpu.sync_copy(dst_hbm.at[pl.ds(base, B)], idst)
        cp1 = pltpu.async_copy(geom_hbm.at[isrc], gsrc, sem)
        cp2 = pltpu.async_copy(geom_hbm.at[idst], gdst, sem)
        cp1.wait()
        cp2.wait()

        @pl.loop(0, GROUPS)
        def _group(g):
            rows = g * L + iota
            xs = plsc.load_gather(gsrc, [rows, full_i(0)])
            ys = plsc.load_gather(gsrc, [rows, full_i(1)])
            zs = plsc.load_gather(gsrc, [rows, full_i(2)])
            ats = plsc.load_gather(gsrc, [rows, full_i(3)]).astype(jnp.int32)
            xd = plsc.load_gather(gdst, [rows, full_i(0)])
            yd = plsc.load_gather(gdst, [rows, full_i(1)])
            zd = plsc.load_gather(gdst, [rows, full_i(2)])
            atd = plsc.load_gather(gdst, [rows, full_i(3)]).astype(jnp.int32)
            ats_v[pl.ds(g * L, L)] = ats
            atd_v[pl.ds(g * L, L)] = atd
            dx = xd - xs
            dy = yd - ys
            dz = zd - zs
            l2 = dx * dx + dy * dy + dz * dz + 1e-12
            inv = _rsqrt(l2)
            x = l2 * inv
            theta = x * (jnp.pi / R_MAX)
            q = (theta * _INV_TWO_PI + 0.5).astype(jnp.int32).astype(jnp.float32)
            th = theta - q * _TWO_PI
            t2 = th * th
            s1 = th * (_SIN_C[0] + t2 * (_SIN_C[1] + t2 * (_SIN_C[2]
                       + t2 * (_SIN_C[3] + t2 * _SIN_C[4]))))
            c1 = (_COS_C[0] + t2 * (_COS_C[1] + t2 * (_COS_C[2]
                  + t2 * (_COS_C[3] + t2 * (_COS_C[4] + t2 * _COS_C[5])))))
            scale = (2.0 / R_MAX) * inv
            two_c1 = 2.0 * c1
            s = s1
            sp = jnp.zeros_like(s1)
            bas = []
            for n in range(NUM_BASIS):
                bas.append(s * scale)
                s, sp = two_c1 * s - sp, s
            for j in range(NUM_BASIS):
                acc = bas[0] * plsc.load_gather(wv, [full_i(8 + j)])
                for n in range(1, NUM_BASIS):
                    acc = acc + bas[n] * plsc.load_gather(
                        wv, [full_i(8 + n * NUM_BASIS + j)])
                plsc.store_scatter(obuf, [rows, full_i(j)], acc)

        cpt0 = pltpu.async_copy(te0_hbm.at[ats_v], otc, sem2)
        cpt1 = pltpu.async_copy(te1_hbm.at[atd_v], otn, sem2)
        cpt0.wait()
        cpt1.wait()

        @pl.loop(0, B, unroll=4)
        def _ileave(e):
            obuf[e, pl.ds(NUM_BASIS, L)] = otc[e, pl.ds(0, L)]
            obuf[e, pl.ds(NUM_BASIS + L, L)] = otc[e, pl.ds(L, L)]
            obuf[e, pl.ds(NUM_BASIS + NUM_TYPES, L)] = otn[e, pl.ds(0, L)]
            obuf[e, pl.ds(NUM_BASIS + NUM_TYPES + L, L)] = otn[e, pl.ds(L, L)]

        pltpu.sync_copy(obuf, out_hbm.at[pl.ds(base, B)])


@jax.jit
def kernel(pos, edge_index, atom_types, type_embeddings, W_basis):
    at32 = atom_types.astype(jnp.int32)
    geom = jnp.concatenate(
        [pos, at32.astype(jnp.float32)[:, None],
         jnp.zeros((N_NODES, GEOM_W - 4), jnp.float32)], axis=1)
    ei = edge_index.astype(jnp.int32)
    src = ei[0]
    dst = ei[1]
    te0 = type_embeddings[0]
    te1 = type_embeddings[1]
    wflat = jnp.concatenate([jnp.zeros((8,), jnp.float32),
                             W_basis.reshape(-1)])

    mesh = plsc.VectorSubcoreMesh(core_axis_name="c", subcore_axis_name="s",
                                  num_cores=NC, num_subcores=NS)
    f = pl.kernel(
        _sc_kernel,
        out_type=jax.ShapeDtypeStruct((N_EDGES, OUT_W), jnp.float32),
        mesh=mesh,
        compiler_params=pltpu.CompilerParams(needs_layout_passes=False,
                                             use_tc_tiling_on_sc=False),
        scratch_types=[
            pltpu.VMEM((B, GEOM_W), jnp.float32),
            pltpu.VMEM((B, GEOM_W), jnp.float32),
            pltpu.VMEM((8 + NUM_BASIS * NUM_BASIS,), jnp.float32),
            pltpu.VMEM((B, OUT_W), jnp.float32),
            pltpu.VMEM((B, NUM_TYPES), jnp.float32),
            pltpu.VMEM((B, NUM_TYPES), jnp.float32),
            pltpu.VMEM((B,), jnp.int32),
            pltpu.VMEM((B,), jnp.int32),
            pltpu.SemaphoreType.DMA,
            pltpu.SemaphoreType.DMA,
            pltpu.VMEM((B,), jnp.int32),
            pltpu.VMEM((B,), jnp.int32),
        ],
    )
    return f(geom, src, dst, te0, te1, wflat)


# speedup vs baseline: 1.0722x; 1.0722x over previous
import jax
import jax.numpy as jnp
from jax import lax
from jax.experimental import pallas as pl
from jax.experimental.pallas import tpu as pltpu
from jax.experimental.pallas import tpu_sc as plsc

N_NODES = 50000
N_EDGES = 800000
NUM_TYPES = 32
NUM_BASIS = 8
R_MAX = 5.0
OUT_W = NUM_BASIS + 2 * NUM_TYPES
OB_W = OUT_W + 1

NC, NS, L = 2, 16, 16
NW = NC * NS
B = 640
NCHUNK = N_EDGES // B
GROUPS = B // L
GEOM_W = 16

_TWO_PI = 6.283185307179586
_INV_TWO_PI = 1.0 / _TWO_PI
_SIN_C = (9.9998458677e-01, -1.6663258204e-01, 8.3123829338e-03,
          -1.9316182196e-04, 2.1732100681e-06)
_COS_C = (9.9999944342e-01, -4.9999558037e-01, 4.1661031574e-02,
          -1.3862743260e-03, 2.4253137751e-05, -2.2193694177e-07)


def _rsqrt(l2):
    i = plsc.bitcast(l2, jnp.int32)
    y = plsc.bitcast(jnp.int32(0x5F3759DF) - (i >> 1), jnp.float32)
    for _ in range(3):
        y = y * (1.5 - 0.5 * l2 * y * y)
    return y


def _sc_kernel(geom_hbm, src_hbm, dst_hbm, te0_hbm, te1_hbm, w_hbm, out_hbm,
               gsrc, gdst, wv, obuf, otc, otn, ats_v, atd_v, sem, sem2, isrc, idst):
    wid = lax.axis_index("s") * NC + lax.axis_index("c")
    pltpu.sync_copy(w_hbm, wv)
    iota = lax.iota(jnp.int32, L)

    def full_i(v):
        return jnp.full((L,), v, jnp.int32)

    @pl.loop(wid, NCHUNK, step=NW)
    def _chunk(c):
        base = c * B
        pltpu.sync_copy(src_hbm.at[pl.ds(base, B)], isrc)
        pltpu.sync_copy(dst_hbm.at[pl.ds(base, B)], idst)
        cp1 = pltpu.async_copy(geom_hbm.at[isrc], gsrc, sem)
        cp2 = pltpu.async_copy(geom_hbm.at[idst], gdst, sem)
        cp1.wait()
        cp2.wait()

        wsp = [plsc.load_gather(wv, [full_i(8 + k)])
               for k in range(NUM_BASIS * NUM_BASIS)]

        @pl.loop(0, GROUPS)
        def _group(g):
            rows = g * L + iota
            xs = plsc.load_gather(gsrc, [rows, full_i(0)])
            ys = plsc.load_gather(gsrc, [rows, full_i(1)])
            zs = plsc.load_gather(gsrc, [rows, full_i(2)])
            ats = plsc.load_gather(gsrc, [rows, full_i(3)]).astype(jnp.int32)
            xd = plsc.load_gather(gdst, [rows, full_i(0)])
            yd = plsc.load_gather(gdst, [rows, full_i(1)])
            zd = plsc.load_gather(gdst, [rows, full_i(2)])
            atd = plsc.load_gather(gdst, [rows, full_i(3)]).astype(jnp.int32)
            ats_v[pl.ds(g * L, L)] = ats
            atd_v[pl.ds(g * L, L)] = atd
            dx = xd - xs
            dy = yd - ys
            dz = zd - zs
            l2 = dx * dx + dy * dy + dz * dz + 1e-12
            inv = _rsqrt(l2)
            x = l2 * inv
            theta = x * (jnp.pi / R_MAX)
            q = (theta * _INV_TWO_PI + 0.5).astype(jnp.int32).astype(jnp.float32)
            th = theta - q * _TWO_PI
            t2 = th * th
            s1 = th * (_SIN_C[0] + t2 * (_SIN_C[1] + t2 * (_SIN_C[2]
                       + t2 * (_SIN_C[3] + t2 * _SIN_C[4]))))
            c1 = (_COS_C[0] + t2 * (_COS_C[1] + t2 * (_COS_C[2]
                  + t2 * (_COS_C[3] + t2 * (_COS_C[4] + t2 * _COS_C[5])))))
            scale = (2.0 / R_MAX) * inv
            two_c1 = 2.0 * c1
            s = s1
            sp = jnp.zeros_like(s1)
            bas = []
            for n in range(NUM_BASIS):
                bas.append(s * scale)
                s, sp = two_c1 * s - sp, s
            for j in range(NUM_BASIS):
                acc = bas[0] * wsp[j]
                for n in range(1, NUM_BASIS):
                    acc = acc + bas[n] * wsp[n * NUM_BASIS + j]
                plsc.store_scatter(obuf, [rows, full_i(j)], acc)

        cpt0 = pltpu.async_copy(te0_hbm.at[ats_v], otc, sem2)
        cpt1 = pltpu.async_copy(te1_hbm.at[atd_v], otn, sem2)
        cpt0.wait()
        cpt1.wait()

        pltpu.sync_copy(obuf.at[:, pl.ds(0, L)],
                        out_hbm.at[pl.ds(base, B), pl.ds(0, L)])
        pltpu.sync_copy(otc, out_hbm.at[pl.ds(base, B),
                                        pl.ds(NUM_BASIS, NUM_TYPES)])
        pltpu.sync_copy(otn, out_hbm.at[pl.ds(base, B),
                                        pl.ds(NUM_BASIS + NUM_TYPES,
                                              NUM_TYPES)])


@jax.jit
def kernel(pos, edge_index, atom_types, type_embeddings, W_basis):
    at32 = atom_types.astype(jnp.int32)
    geom = jnp.concatenate(
        [pos, at32.astype(jnp.float32)[:, None],
         jnp.zeros((N_NODES, GEOM_W - 4), jnp.float32)], axis=1)
    ei = edge_index.astype(jnp.int32)
    src = ei[0]
    dst = ei[1]
    te0 = type_embeddings[0]
    te1 = type_embeddings[1]
    wflat = jnp.concatenate([jnp.zeros((8,), jnp.float32),
                             W_basis.reshape(-1)])

    mesh = plsc.VectorSubcoreMesh(core_axis_name="c", subcore_axis_name="s",
                                  num_cores=NC, num_subcores=NS)
    f = pl.kernel(
        _sc_kernel,
        out_type=jax.ShapeDtypeStruct((N_EDGES, OUT_W), jnp.float32),
        mesh=mesh,
        compiler_params=pltpu.CompilerParams(needs_layout_passes=False,
                                             use_tc_tiling_on_sc=False),
        scratch_types=[
            pltpu.VMEM((B, GEOM_W), jnp.float32),
            pltpu.VMEM((B, GEOM_W), jnp.float32),
            pltpu.VMEM((8 + NUM_BASIS * NUM_BASIS,), jnp.float32),
            pltpu.VMEM((B, 17), jnp.float32),
            pltpu.VMEM((B, NUM_TYPES), jnp.float32),
            pltpu.VMEM((B, NUM_TYPES), jnp.float32),
            pltpu.VMEM((B,), jnp.int32),
            pltpu.VMEM((B,), jnp.int32),
            pltpu.SemaphoreType.DMA,
            pltpu.SemaphoreType.DMA,
            pltpu.VMEM((B,), jnp.int32),
            pltpu.VMEM((B,), jnp.int32),
        ],
    )
    return f(geom, src, dst, te0, te1, wflat)


# speedup vs baseline: 1.0829x; 1.0100x over previous
import jax
import jax.numpy as jnp
from jax import lax
from jax.experimental import pallas as pl
from jax.experimental.pallas import tpu as pltpu
from jax.experimental.pallas import tpu_sc as plsc

N_NODES = 50000
N_EDGES = 800000
NUM_TYPES = 32
NUM_BASIS = 8
R_MAX = 5.0
OUT_W = NUM_BASIS + 2 * NUM_TYPES

NC, NS, L = 2, 16, 16
NW = NC * NS
B = 400
NCHUNK = N_EDGES // B
GROUPS = B // L
GEOM_W = 16
NK = -(-NCHUNK // NW)
NPAIR = (NK + 3) // 2 + 1

_TWO_PI = 6.283185307179586
_INV_TWO_PI = 1.0 / _TWO_PI
_SIN_C = (9.9998458677e-01, -1.6663258204e-01, 8.3123829338e-03,
          -1.9316182196e-04, 2.1732100681e-06)
_COS_C = (9.9999944342e-01, -4.9999558037e-01, 4.1661031574e-02,
          -1.3862743260e-03, 2.4253137751e-05, -2.2193694177e-07)


def _rsqrt(l2):
    i = plsc.bitcast(l2, jnp.int32)
    y = plsc.bitcast(jnp.int32(0x5F3759DF) - (i >> 1), jnp.float32)
    for _ in range(3):
        y = y * (1.5 - 0.5 * l2 * y * y)
    return y


NSET = 13


def _sc_kernel(geom_hbm, src_hbm, dst_hbm, te0_hbm, te1_hbm, w_hbm, out_hbm,
               wv, *bufs):
    sets = [bufs[:NSET], bufs[NSET:]]
    wid = lax.axis_index("s") * NC + lax.axis_index("c")
    pltpu.sync_copy(w_hbm, wv)
    iota = lax.iota(jnp.int32, L)

    def full_i(v):
        return jnp.full((L,), v, jnp.int32)

    wsp = [plsc.load_gather(wv, [full_i(8 + k)])
           for k in range(NUM_BASIS * NUM_BASIS)]

    def idx_fire(S, c):
        base = c * B
        pltpu.async_copy(src_hbm.at[pl.ds(base, B)], S[0], S[9])
        pltpu.async_copy(dst_hbm.at[pl.ds(base, B)], S[1], S[9])

    def idx_wait(S, c):
        base = c * B
        pltpu.make_async_copy(src_hbm.at[pl.ds(base, B)], S[0], S[9]).wait()
        pltpu.make_async_copy(dst_hbm.at[pl.ds(base, B)], S[1], S[9]).wait()

    def geom_fire(S):
        pltpu.async_copy(geom_hbm.at[S[0]], S[2], S[10])
        pltpu.async_copy(geom_hbm.at[S[1]], S[3], S[10])

    def geom_wait(S):
        pltpu.make_async_copy(geom_hbm.at[S[0]], S[2], S[10]).wait()
        pltpu.make_async_copy(geom_hbm.at[S[1]], S[3], S[10]).wait()

    def te_fire(S):
        pltpu.async_copy(te0_hbm.at[S[4]], S[7], S[11])
        pltpu.async_copy(te1_hbm.at[S[5]], S[8], S[11])

    def te_wait(S):
        pltpu.make_async_copy(te0_hbm.at[S[4]], S[7], S[11]).wait()
        pltpu.make_async_copy(te1_hbm.at[S[5]], S[8], S[11]).wait()

    def wb_descr(S, c):
        base = c * B
        return [
            (S[6].at[:, pl.ds(0, L)],
             out_hbm.at[pl.ds(base, B), pl.ds(0, L)], S[12]),
            (S[7], out_hbm.at[pl.ds(base, B),
                              pl.ds(NUM_BASIS, NUM_TYPES)], S[12]),
            (S[8], out_hbm.at[pl.ds(base, B),
                              pl.ds(NUM_BASIS + NUM_TYPES, NUM_TYPES)],
             S[12]),
        ]

    def wb_fire(S, c):
        for a, b, sm in wb_descr(S, c):
            pltpu.async_copy(a, b, sm)

    def wb_wait(S, c):
        for a, b, sm in wb_descr(S, c):
            pltpu.make_async_copy(a, b, sm).wait()

    def compute(S):
        gsrc, gdst, ats_v, atd_v, obasis = S[2], S[3], S[4], S[5], S[6]

        @pl.loop(0, GROUPS)
        def _group(g):
            rows = g * L + iota
            xs = plsc.load_gather(gsrc, [rows, full_i(0)])
            ys = plsc.load_gather(gsrc, [rows, full_i(1)])
            zs = plsc.load_gather(gsrc, [rows, full_i(2)])
            ats = plsc.load_gather(gsrc, [rows, full_i(3)]).astype(jnp.int32)
            xd = plsc.load_gather(gdst, [rows, full_i(0)])
            yd = plsc.load_gather(gdst, [rows, full_i(1)])
            zd = plsc.load_gather(gdst, [rows, full_i(2)])
            atd = plsc.load_gather(gdst, [rows, full_i(3)]).astype(jnp.int32)
            ats_v[pl.ds(g * L, L)] = ats
            atd_v[pl.ds(g * L, L)] = atd
            dx = xd - xs
            dy = yd - ys
            dz = zd - zs
            l2 = dx * dx + dy * dy + dz * dz + 1e-12
            inv = _rsqrt(l2)
            x = l2 * inv
            theta = x * (jnp.pi / R_MAX)
            q = (theta * _INV_TWO_PI + 0.5).astype(jnp.int32).astype(
                jnp.float32)
            th = theta - q * _TWO_PI
            t2 = th * th
            s1 = th * (_SIN_C[0] + t2 * (_SIN_C[1] + t2 * (_SIN_C[2]
                       + t2 * (_SIN_C[3] + t2 * _SIN_C[4]))))
            c1 = (_COS_C[0] + t2 * (_COS_C[1] + t2 * (_COS_C[2]
                  + t2 * (_COS_C[3] + t2 * (_COS_C[4] + t2 * _COS_C[5])))))
            scale = (2.0 / R_MAX) * inv
            two_c1 = 2.0 * c1
            s = s1
            sp = jnp.zeros_like(s1)
            bas = []
            for n in range(NUM_BASIS):
                bas.append(s * scale)
                s, sp = two_c1 * s - sp, s
            for j in range(NUM_BASIS):
                acc = bas[0] * wsp[j]
                for n in range(1, NUM_BASIS):
                    acc = acc + bas[n] * wsp[n * NUM_BASIS + j]
                plsc.store_scatter(obasis, [rows, full_i(j)], acc)

    idx_fire(sets[0], wid)

    @pl.loop(0, NPAIR)
    def _pair(kp):
        for p in (0, 1):
            k = 2 * kp + p
            S = sets[p]
            T = sets[1 - p]
            c = wid + k * NW
            cm1 = c - NW
            cm2 = c - 2 * NW

            @pl.when(c < NCHUNK)
            def _():
                idx_wait(S, c)
                geom_fire(S)

            @pl.when(jnp.logical_and(cm1 >= 0, cm1 < NCHUNK))
            def _():
                te_wait(T)
                wb_fire(T, cm1)

            @pl.when(jnp.logical_and(cm2 >= 0, cm2 < NCHUNK))
            def _():
                wb_wait(S, cm2)

            @pl.when(c < NCHUNK)
            def _():
                geom_wait(S)
                compute(S)
                te_fire(S)

            @pl.when(c + NW < NCHUNK)
            def _():
                idx_fire(T, c + NW)


@jax.jit
def kernel(pos, edge_index, atom_types, type_embeddings, W_basis):
    at32 = atom_types.astype(jnp.int32)
    geom = jnp.concatenate(
        [pos, at32.astype(jnp.float32)[:, None],
         jnp.zeros((N_NODES, GEOM_W - 4), jnp.float32)], axis=1)
    ei = edge_index.astype(jnp.int32)
    src = ei[0]
    dst = ei[1]
    te0 = type_embeddings[0]
    te1 = type_embeddings[1]
    wflat = jnp.concatenate([jnp.zeros((8,), jnp.float32),
                             W_basis.reshape(-1)])

    def one_set():
        return [
            pltpu.VMEM((B,), jnp.int32),
            pltpu.VMEM((B,), jnp.int32),
            pltpu.VMEM((B, GEOM_W), jnp.float32),
            pltpu.VMEM((B, GEOM_W), jnp.float32),
            pltpu.VMEM((B,), jnp.int32),
            pltpu.VMEM((B,), jnp.int32),
            pltpu.VMEM((B, 17), jnp.float32),
            pltpu.VMEM((B, NUM_TYPES), jnp.float32),
            pltpu.VMEM((B, NUM_TYPES), jnp.float32),
            pltpu.SemaphoreType.DMA,
            pltpu.SemaphoreType.DMA,
            pltpu.SemaphoreType.DMA,
            pltpu.SemaphoreType.DMA,
        ]

    mesh = plsc.VectorSubcoreMesh(core_axis_name="c", subcore_axis_name="s",
                                  num_cores=NC, num_subcores=NS)
    f = pl.kernel(
        _sc_kernel,
        out_type=jax.ShapeDtypeStruct((N_EDGES, OUT_W), jnp.float32),
        mesh=mesh,
        compiler_params=pltpu.CompilerParams(needs_layout_passes=False,
                                             use_tc_tiling_on_sc=False),
        scratch_types=[pltpu.VMEM((8 + NUM_BASIS * NUM_BASIS,), jnp.float32)]
        + one_set() + one_set(),
    )
    return f(geom, src, dst, te0, te1, wflat)
